# baseline (speedup 1.0000x reference)
import jax
import jax.numpy as jnp
from jax import lax
from jax.experimental import pallas as pl
from jax.experimental.pallas import tpu as pltpu
from jax.experimental.pallas import tpu_sc as plsc

NC = 2
NS = 16
NW = NC * NS
LANES = 16

BATCH = 16384
EMBED_DIM = 64
B_PER_W = BATCH // NW
CHUNK = 128
N_CHUNKS = B_PER_W // CHUNK

NROWS = 1000000
RANGE = 32768
SUB = 4096
NSUB = RANGE // SUB
MAXSTART = 999936 - SUB
TAIL0 = 999936
ECAP = 128
MYCAP = 768
TRASH = BATCH


def _k1_body(user_ref, item_ref, uembT_ref, iembT_ref, stage_u_ref,
             stage_i_ref,
             user_v, item_v, chunk_v, staging, myslots, el_slots, el_locs,
             sem):
    wid = lax.axis_index("s") * NC + lax.axis_index("c")
    lane = lax.iota(jnp.int32, LANES)

    pltpu.sync_copy(user_ref, user_v)
    pltpu.sync_copy(item_ref, item_v)

    def one_table(idx_v, embT_ref, stage_ref):
        def scanA(g, cnt):
            k16 = g * LANES + lane
            u16 = idx_v[pl.ds(g * LANES, LANES)]
            m = (u16 >> 15) == wid
            off = jnp.minimum(cnt, MYCAP - LANES)
            plsc.store_compressed(myslots.at[pl.ds(off, LANES)], k16, mask=m)
            npop = plsc.all_reduce_population_count(m)[0]
            return cnt + npop

        cnt = lax.fori_loop(0, BATCH // LANES, scanA, 0)

        for e in range(NSUB):
            full_row = jnp.full((LANES,), TRASH, jnp.int32)
            zero_row = jnp.full((LANES,), 0, jnp.int32)
            for gg in range(ECAP // LANES):
                el_slots[e, pl.ds(gg * LANES, LANES)] = full_row
                el_locs[e, pl.ds(gg * LANES, LANES)] = zero_row

        def scanB(e):
            s_e = jnp.minimum(wid * RANGE + e * SUB, MAXSTART)

            def body(g, ecnt):
                pos = g * LANES + lane
                k16 = plsc.load_gather(myslots, [pos]) & (BATCH - 1)
                u16 = plsc.load_gather(idx_v, [k16])
                m = ((pos < cnt) & (((u16 >> 12) & 7) == e)
                     & (u16 < TAIL0) & ((u16 >> 15) == wid))
                off = jnp.minimum(ecnt, ECAP - LANES)
                plsc.store_compressed(
                    el_slots.at[e].at[pl.ds(off, LANES)], k16, mask=m)
                plsc.store_compressed(
                    el_locs.at[e].at[pl.ds(off, LANES)], u16 - s_e, mask=m)
                npop = plsc.all_reduce_population_count(m)[0]
                return ecnt + npop

            lax.fori_loop(0, MYCAP // LANES, body, 0)

        for e in range(NSUB):
            scanB(e)

        for e in range(NSUB):
            s_e = jnp.minimum(wid * RANGE + e * SUB, MAXSTART)
            s_e = pl.multiple_of(s_e, 128)

            def do_c8(c8, _):
                pltpu.async_copy(
                    embT_ref.at[c8, :, pl.ds(s_e, SUB)], chunk_v, sem).wait()

                def grp(g, _):
                    loc16 = el_locs[e, pl.ds(g * LANES, LANES)]
                    base_col = c8 * 8

                    def dims(d, _):
                        val = plsc.load_gather(chunk_v, [jnp.full(
                            (LANES,), d, jnp.int32), loc16])
                        row16 = g * LANES + lane
                        plsc.store_scatter(
                            staging, [row16, jnp.full((LANES,), base_col + d,
                                                      jnp.int32)], val)
                        return 0

                    lax.fori_loop(0, 8, dims, 0)
                    return 0

                lax.fori_loop(0, ECAP // LANES, grp, 0)
                return 0

            lax.fori_loop(0, 8, do_c8, 0)

            pltpu.async_copy(staging, stage_ref.at[el_slots.at[e]],
                             sem).wait()

    one_table(user_v, uembT_ref, stage_u_ref)
    one_table(item_v, iembT_ref, stage_i_ref)


def _k2_body(stage_u_ref, stage_i_ref, uorig_ref, iorig_ref, gb_ref,
             ubias_ref, ibias_ref, tailu_ref, taili_ref, out_ref,
             idxo_u, idxo_i, pass_u, pass_i, tailu_v, taili_v, ub_v, ib_v,
             gb_v, out_v, sem, bsem):
    wid = lax.axis_index("s") * NC + lax.axis_index("c")
    lane = lax.iota(jnp.int32, LANES)

    pltpu.sync_copy(uorig_ref.at[wid], idxo_u)
    pltpu.sync_copy(iorig_ref.at[wid], idxo_i)
    pltpu.sync_copy(gb_ref, gb_v)
    pltpu.sync_copy(tailu_ref, tailu_v)
    pltpu.sync_copy(taili_ref, taili_v)

    bias_copies = []
    for c in range(N_CHUNKS):
        sl = pl.ds(c * CHUNK, CHUNK)
        bias_copies.append(
            pltpu.async_copy(ubias_ref.at[idxo_u.at[c]], ub_v.at[sl], bsem))
        bias_copies.append(
            pltpu.async_copy(ibias_ref.at[idxo_i.at[c]], ib_v.at[sl], bsem))
    for cp in bias_copies:
        cp.wait()

    gb_vec = gb_v[...]
    NPASS = 4
    PROWS = B_PER_W // NPASS

    def do_pass(p, _):
        row0 = wid * B_PER_W + p * PROWS
        row0 = pl.multiple_of(row0, 8)
        pltpu.sync_copy(stage_u_ref.at[pl.ds(row0, PROWS)], pass_u)
        pltpu.sync_copy(stage_i_ref.at[pl.ds(row0, PROWS)], pass_i)

        def group(g, _):
            base = p * PROWS + g * LANES
            row16 = g * LANES + lane
            iu = idxo_u[base >> 7, pl.ds(base & 127, LANES)]
            ii = idxo_i[base >> 7, pl.ds(base & 127, LANES)]
            um = iu >= TAIL0
            im = ii >= TAIL0
            ur_t = (iu - TAIL0) & 63
            ir_t = (ii - TAIL0) & 63

            def dot_step(j, acc):
                col = (j & 48) + ((lane + j) & 15)
                u = plsc.load_gather(pass_u, [row16, col])
                v = plsc.load_gather(pass_i, [row16, col])
                ut = plsc.load_gather(tailu_v, [ur_t, col])
                vt = plsc.load_gather(taili_v, [ir_t, col])
                u = jnp.where(um, ut, u)
                v = jnp.where(im, vt, v)
                return acc + u * v

            acc0 = gb_vec + ub_v[pl.ds(base, LANES)] + ib_v[pl.ds(base, LANES)]
            acc = lax.fori_loop(0, EMBED_DIM, dot_step, acc0)
            out_v[pl.ds(base, LANES)] = acc
            return 0

        lax.fori_loop(0, PROWS // LANES, group, 0)
        return 0

    lax.fori_loop(0, NPASS, do_pass, 0)
    pltpu.sync_copy(out_v, out_ref.at[wid])


@jax.jit
def _mf_sc(user, item, uembT3, iembT3, uorig3, iorig3, gb16, user_bias,
           item_bias, tail_u, tail_i):
    mesh = plsc.VectorSubcoreMesh(core_axis_name="c", subcore_axis_name="s")
    cparams = pltpu.CompilerParams(needs_layout_passes=False)

    k1 = pl.kernel(
        _k1_body,
        out_type=(jax.ShapeDtypeStruct((BATCH + 8, 2 * EMBED_DIM),
                                       jnp.float32),
                  jax.ShapeDtypeStruct((BATCH + 8, 2 * EMBED_DIM),
                                       jnp.float32)),
        mesh=mesh,
        compiler_params=cparams,
        scratch_types=[
            pltpu.VMEM((BATCH,), jnp.int32),
            pltpu.VMEM((BATCH,), jnp.int32),
            pltpu.VMEM((8, SUB), jnp.float32),
            pltpu.VMEM((ECAP, 2 * EMBED_DIM), jnp.float32),
            pltpu.VMEM((MYCAP,), jnp.int32),
            pltpu.VMEM((NSUB, ECAP), jnp.int32),
            pltpu.VMEM((NSUB, ECAP), jnp.int32),
            pltpu.SemaphoreType.DMA,
        ],
    )
    stage_u, stage_i = k1(user, item, uembT3, iembT3)

    k2 = pl.kernel(
        _k2_body,
        out_type=jax.ShapeDtypeStruct((NW, B_PER_W), jnp.float32),
        mesh=mesh,
        compiler_params=cparams,
        scratch_types=[
            pltpu.VMEM((N_CHUNKS, CHUNK), jnp.int32),
            pltpu.VMEM((N_CHUNKS, CHUNK), jnp.int32),
            pltpu.VMEM((B_PER_W // 4, 2 * EMBED_DIM), jnp.float32),
            pltpu.VMEM((B_PER_W // 4, 2 * EMBED_DIM), jnp.float32),
            pltpu.VMEM((EMBED_DIM, EMBED_DIM), jnp.float32),
            pltpu.VMEM((EMBED_DIM, EMBED_DIM), jnp.float32),
            pltpu.VMEM((B_PER_W,), jnp.float32),
            pltpu.VMEM((B_PER_W,), jnp.float32),
            pltpu.VMEM((LANES,), jnp.float32),
            pltpu.VMEM((B_PER_W,), jnp.float32),
            pltpu.SemaphoreType.DMA,
            pltpu.SemaphoreType.DMA,
        ],
    )
    return k2(stage_u, stage_i, uorig3, iorig3, gb16, user_bias, item_bias,
              tail_u, tail_i)


def kernel(user, item, user_emb, item_emb, global_bias, user_bias, item_bias):
    uembT3 = user_emb.T.reshape(8, 8, NROWS)
    iembT3 = item_emb.T.reshape(8, 8, NROWS)
    uorig3 = user.reshape(NW, N_CHUNKS, CHUNK)
    iorig3 = item.reshape(NW, N_CHUNKS, CHUNK)
    tail_u = user_emb[TAIL0:, :]
    tail_i = item_emb[TAIL0:, :]
    gb16 = jnp.broadcast_to(global_bias, (LANES,))
    out = _mf_sc(user, item, uembT3, iembT3, uorig3, iorig3, gb16,
                 user_bias, item_bias, tail_u, tail_i)
    return out.reshape(BATCH)


# speedup vs baseline: 3.5217x; 3.5217x over previous
import jax
import jax.numpy as jnp
from jax import lax
from jax.experimental import pallas as pl
from jax.experimental.pallas import tpu as pltpu
from jax.experimental.pallas import tpu_sc as plsc

NC = 2
NS = 16
NW = NC * NS
LANES = 16

BATCH = 16384
EMBED_DIM = 64
SLAB = 8
B_PER_W = BATCH // NW
CHUNK = 128
N_CHUNKS = B_PER_W // CHUNK
RCH = 32
NRND = B_PER_W // RCH


def _mf_body(uorig_ref, iorig_ref, uemb_ref, iemb_ref, gb_ref, ubias_ref,
             ibias_ref, out_ref,
             idxo_u, idxo_i, u_slabs, i_slabs, ub_v, ib_v, gb_v, out_v,
             sem, bsem):
    wid = lax.axis_index("s") * NC + lax.axis_index("c")

    pltpu.sync_copy(uorig_ref.at[wid], idxo_u)
    pltpu.sync_copy(iorig_ref.at[wid], idxo_i)
    pltpu.sync_copy(gb_ref, gb_v)

    bias_copies = []
    for c in range(N_CHUNKS):
        sl = pl.ds(c * CHUNK, CHUNK)
        bias_copies.append(
            pltpu.async_copy(ubias_ref.at[idxo_u.at[c]], ub_v.at[sl], bsem))
        bias_copies.append(
            pltpu.async_copy(ibias_ref.at[idxo_i.at[c]], ib_v.at[sl], bsem))
    for cp in bias_copies:
        cp.wait()

    lane = lax.iota(jnp.int32, LANES)
    gb_vec = gb_v[...]

    def round_body(r, _):
        def enq_group(g, _):
            base = r * RCH + g * LANES
            iu = idxo_u[base >> 7, pl.ds(base & 127, LANES)]
            ii = idxo_i[base >> 7, pl.ds(base & 127, LANES)]
            su_vec = iu >> 3
            si_vec = (ii >> 3) * SLAB
            for t in range(LANES):
                e_loc = g * LANES + t
                pltpu.async_copy(uemb_ref.at[su_vec[t]], u_slabs.at[e_loc],
                                 sem)
                si = pl.multiple_of(si_vec[t], SLAB)
                pltpu.async_copy(iemb_ref.at[pl.ds(si, SLAB)],
                                 i_slabs.at[e_loc], sem)
            return 0

        lax.fori_loop(0, RCH // LANES, enq_group, 0)

        pltpu.make_async_copy(uemb_ref.at[pl.ds(0, RCH)], u_slabs, sem).wait()
        pltpu.make_async_copy(uemb_ref.at[pl.ds(0, RCH)], i_slabs, sem).wait()

        def group(g, _):
            base = r * RCH + g * LANES
            iu = idxo_u[base >> 7, pl.ds(base & 127, LANES)]
            ii = idxo_i[base >> 7, pl.ds(base & 127, LANES)]
            urow = iu & 7
            irow = ii & 7
            p_vec = g * LANES + lane

            def dot_step(j, acc):
                col = (j & 48) + ((lane + j) & 15)
                u = plsc.load_gather(u_slabs, [p_vec, urow, col])
                v = plsc.load_gather(i_slabs, [p_vec, irow, col])
                return acc + u * v

            acc0 = gb_vec + ub_v[pl.ds(base, LANES)] + ib_v[pl.ds(base, LANES)]
            acc = lax.fori_loop(0, EMBED_DIM, dot_step, acc0)
            out_v[pl.ds(base, LANES)] = acc
            return 0

        lax.fori_loop(0, RCH // LANES, group, 0)
        return 0

    lax.fori_loop(0, NRND, round_body, 0)

    pltpu.sync_copy(out_v, out_ref.at[wid])


@jax.jit
def _mf_sc(uorig3, iorig3, uemb3, iemb3, gb16, user_bias, item_bias):
    mesh = plsc.VectorSubcoreMesh(core_axis_name="c", subcore_axis_name="s")
    k = pl.kernel(
        _mf_body,
        out_type=jax.ShapeDtypeStruct((NW, B_PER_W), jnp.float32),
        mesh=mesh,
        compiler_params=pltpu.CompilerParams(needs_layout_passes=False),
        scratch_types=[
            pltpu.VMEM((N_CHUNKS, CHUNK), jnp.int32),
            pltpu.VMEM((N_CHUNKS, CHUNK), jnp.int32),
            pltpu.VMEM((RCH, SLAB, EMBED_DIM), jnp.float32),
            pltpu.VMEM((RCH, SLAB, EMBED_DIM), jnp.float32),
            pltpu.VMEM((B_PER_W,), jnp.float32),
            pltpu.VMEM((B_PER_W,), jnp.float32),
            pltpu.VMEM((LANES,), jnp.float32),
            pltpu.VMEM((B_PER_W,), jnp.float32),
            pltpu.SemaphoreType.DMA,
            pltpu.SemaphoreType.DMA,
        ],
    )
    return k(uorig3, iorig3, uemb3, iemb3, gb16, user_bias, item_bias)


def kernel(user, item, user_emb, item_emb, global_bias, user_bias, item_bias):
    uorig3 = user.reshape(NW, N_CHUNKS, CHUNK)
    iorig3 = item.reshape(NW, N_CHUNKS, CHUNK)
    uemb3 = user_emb.reshape(-1, SLAB, EMBED_DIM)
    gb16 = jnp.broadcast_to(global_bias, (LANES,))
    out = _mf_sc(uorig3, iorig3, uemb3, item_emb, gb16, user_bias, item_bias)
    return out.reshape(BATCH)
